# baseline (speedup 1.0000x reference)
import dataclasses
import functools
import math

import jax
import jax.numpy as jnp
import numpy as np
from jax import lax
from jax.experimental import pallas as pl
from jax.experimental.pallas import tpu as pltpu
from jax.experimental.pallas import tpu_sc as plsc

_D = 768
_S = 2048
_B = 4
_NC = 2
_NS = 16
_NW = _NC * _NS
_S_PER_W = _S // _NW
_CHUNK = 16
_NBLK = (_B * _S_PER_W) // _CHUNK
_NBUF = 4
_SC_PER_W = _S_PER_W // _CHUNK
_LANES = 16


def _pe_table():
    positions = np.arange(_S, dtype=np.float32)[:, None]
    denominator = np.exp(
        np.arange(0, _D, 2, dtype=np.float32) * (-math.log(10000.0) / _D)
    )
    enc = np.zeros((_S, _D), dtype=np.float32)
    enc[:, 0::2] = np.sin(positions * denominator)
    enc[:, 1::2] = np.cos(positions * denominator)
    return enc


def _pe_table_packed():
    pe = _pe_table()
    sh = pe.reshape(_S, _D // 32, 2, 16).transpose(0, 1, 3, 2)
    flat = np.ascontiguousarray(sh).reshape(_S * _D)
    bits = flat.view(np.uint32)
    bf16 = ((bits + 0x7FFF + ((bits >> 16) & 1)) >> 16).astype(np.uint16)
    return bf16.view(np.uint32).view(np.int32)


_PE = _pe_table_packed()
_PE_DEV = None


def _pe_on_device():
    global _PE_DEV
    if _PE_DEV is None:
        _PE_DEV = jax.device_put(_PE)
    return _PE_DEV


@jax.jit
def _embed(idx, table, pe):
    mesh = plsc.VectorSubcoreMesh(core_axis_name="c", subcore_axis_name="s")
    cp = pltpu.CompilerParams()
    if "needs_layout_passes" in pltpu.CompilerParams.__dataclass_fields__:
        cp = dataclasses.replace(cp, needs_layout_passes=False)

    @functools.partial(
        pl.kernel,
        out_type=jax.ShapeDtypeStruct((_B * _S, _D), jnp.float32),
        mesh=mesh,
        compiler_params=cp,
        scratch_types=[
            pltpu.VMEM((_B, _S_PER_W), jnp.int32),
            pltpu.VMEM((_S_PER_W * _D // 2,), jnp.int32),
            pltpu.VMEM((_NBUF, _CHUNK, _D), jnp.float32),
            pltpu.SemaphoreType.DMA,
            pltpu.SemaphoreType.DMA,
            pltpu.SemaphoreType.DMA,
            pltpu.SemaphoreType.DMA,
            pltpu.SemaphoreType.DMA,
            pltpu.SemaphoreType.DMA,
            pltpu.SemaphoreType.DMA,
            pltpu.SemaphoreType.DMA,
            pltpu.SemaphoreType.DMA,
            pltpu.SemaphoreType.DMA,
        ],
    )
    def run(idx_hbm, table_hbm, pe_hbm, out_hbm, idx_v, pe_v, rows_v,
            g0, g1, g2, g3, o0, o1, o2, o3, pe_sem, i_sem):
        g_sem = (g0, g1, g2, g3)
        o_sem = (o0, o1, o2, o3)
        wid = lax.axis_index("s") * _NC + lax.axis_index("c")
        s_base = wid * _S_PER_W

        pe_base = pl.multiple_of(s_base * (_D // 2), 8)
        pltpu.async_copy(pe_hbm.at[pl.ds(pe_base, _S_PER_W * _D // 2)],
                         pe_v, pe_sem)
        for bb in range(_B):
            pltpu.async_copy(idx_hbm.at[bb, pl.ds(s_base, _S_PER_W)],
                             idx_v.at[bb], i_sem)
        for bb in range(_B):
            pltpu.make_async_copy(
                idx_hbm.at[0, pl.ds(0, _S_PER_W)], idx_v.at[0], i_sem).wait()

        def start_gather(i, buf):
            c = i >> 2
            b = i & 3
            pltpu.async_copy(
                table_hbm.at[idx_v.at[b, pl.ds(c * _CHUNK, _CHUNK)]],
                rows_v.at[buf], g_sem[buf])

        _LOOK = 2
        for i in range(_LOOK):
            start_gather(i, i)
        pltpu.make_async_copy(
            pe_hbm.at[pl.ds(0, _S_PER_W * _D // 2)], pe_v, pe_sem).wait()

        @pl.loop(0, _NBLK, step=_NBUF)
        def _(i0):
            for u in range(_NBUF):
                i = i0 + u
                buf = u
                nbuf = (u + 2) % _NBUF
                c = i >> 2
                b = i & 3

                @pl.when((i >= 2) & (i < _NBLK - 2))
                def _():
                    pltpu.make_async_copy(
                        rows_v.at[nbuf], out_hbm.at[pl.ds(0, _CHUNK)],
                        o_sem[nbuf]).wait()

                @pl.when(i < _NBLK - 2)
                def _():
                    start_gather(i + 2, nbuf)

                pltpu.make_async_copy(
                    table_hbm.at[idx_v.at[0, pl.ds(0, _CHUNK)]],
                    rows_v.at[buf], g_sem[buf]).wait()

                @plsc.parallel_loop(0, _CHUNK, unroll=2)
                def _(r):
                    pe_off = (c * _CHUNK + r) * (_D // 2)
                    for j in range(_D // 32):
                        off = pl.multiple_of(pe_off + j * 16, 8)
                        ab32 = pe_v[pl.ds(off, 16)]
                        ab = plsc.bitcast(ab32, jnp.bfloat16)
                        lo, hi = plsc.unpack(
                            ab, format=plsc.PackFormat.INTERLEAVED)
                        plsc.addupdate(
                            rows_v.at[buf, r, pl.ds(j * 32, _LANES)], lo)
                        plsc.addupdate(
                            rows_v.at[buf, r, pl.ds(j * 32 + 16, _LANES)], hi)

                row0 = b * _S + s_base + c * _CHUNK
                pltpu.async_copy(
                    rows_v.at[buf], out_hbm.at[pl.ds(row0, _CHUNK)],
                    o_sem[buf])

        for buf in range(_NBUF):
            pltpu.make_async_copy(
                rows_v.at[buf], out_hbm.at[pl.ds(0, _CHUNK)], o_sem[buf]
            ).wait()

    return run(idx, table, pe)


def kernel(x, table):
    idx = x.astype(jnp.int32)
    out = _embed(idx, table, _pe_on_device())
    return out.reshape(_B, _S, _D)


# speedup vs baseline: 1.0457x; 1.0457x over previous
import dataclasses
import functools
import math

import jax
import jax.numpy as jnp
import numpy as np
from jax import lax
from jax.experimental import pallas as pl
from jax.experimental.pallas import tpu as pltpu
from jax.experimental.pallas import tpu_sc as plsc

_D = 768
_S = 2048
_B = 4
_NC = 2
_NS = 16
_NW = _NC * _NS
_S_PER_W = _S // _NW
_CHUNK = 32
_NBLK = (_B * _S_PER_W) // _CHUNK
_NBUF = 4
_LANES = 16


def _pe_table():
    positions = np.arange(_S, dtype=np.float32)[:, None]
    denominator = np.exp(
        np.arange(0, _D, 2, dtype=np.float32) * (-math.log(10000.0) / _D)
    )
    enc = np.zeros((_S, _D), dtype=np.float32)
    enc[:, 0::2] = np.sin(positions * denominator)
    enc[:, 1::2] = np.cos(positions * denominator)
    return enc


def _pe_table_packed():
    pe = _pe_table()
    sh = pe.reshape(_S, _D // 32, 2, 16).transpose(0, 1, 3, 2)
    flat = np.ascontiguousarray(sh).reshape(_S * _D)
    bits = flat.view(np.uint32)
    bf16 = ((bits + 0x7FFF + ((bits >> 16) & 1)) >> 16).astype(np.uint16)
    return bf16.view(np.uint32).view(np.int32)


_PE = _pe_table_packed()
_PE_DEV = None


def _pe_on_device():
    global _PE_DEV
    if _PE_DEV is None:
        _PE_DEV = jax.device_put(_PE)
    return _PE_DEV


@jax.jit
def _embed(idx, table, pe):
    mesh = plsc.VectorSubcoreMesh(core_axis_name="c", subcore_axis_name="s")
    cp = pltpu.CompilerParams()
    if "needs_layout_passes" in pltpu.CompilerParams.__dataclass_fields__:
        cp = dataclasses.replace(cp, needs_layout_passes=False)

    @functools.partial(
        pl.kernel,
        out_type=jax.ShapeDtypeStruct((_B * _S, _D), jnp.float32),
        mesh=mesh,
        compiler_params=cp,
        scratch_types=[
            pltpu.VMEM((_B, _S_PER_W), jnp.int32),
            pltpu.VMEM((_S_PER_W * _D // 2,), jnp.int32),
            pltpu.VMEM((_NBUF, _CHUNK, _D), jnp.float32),
            pltpu.SemaphoreType.DMA,
            pltpu.SemaphoreType.DMA,
            pltpu.SemaphoreType.DMA,
            pltpu.SemaphoreType.DMA,
            pltpu.SemaphoreType.DMA,
            pltpu.SemaphoreType.DMA,
            pltpu.SemaphoreType.DMA,
            pltpu.SemaphoreType.DMA,
            pltpu.SemaphoreType.DMA,
            pltpu.SemaphoreType.DMA,
        ],
    )
    def run(idx_hbm, table_hbm, pe_hbm, out_hbm, idx_v, pe_v, rows_v,
            g0, g1, g2, g3, o0, o1, o2, o3, pe_sem, i_sem):
        g_sem = (g0, g1, g2, g3)
        o_sem = (o0, o1, o2, o3)
        wid = lax.axis_index("s") * _NC + lax.axis_index("c")
        s_base = wid * _S_PER_W

        pe_base = pl.multiple_of(s_base * (_D // 2), 8)
        pltpu.async_copy(pe_hbm.at[pl.ds(pe_base, _S_PER_W * _D // 2)],
                         pe_v, pe_sem)
        for bb in range(_B):
            pltpu.async_copy(idx_hbm.at[bb, pl.ds(s_base, _S_PER_W)],
                             idx_v.at[bb], i_sem)
        for bb in range(_B):
            pltpu.make_async_copy(
                idx_hbm.at[0, pl.ds(0, _S_PER_W)], idx_v.at[0], i_sem).wait()

        def start_gather(i, buf):
            c = i >> 2
            b = i & 3
            pltpu.async_copy(
                table_hbm.at[idx_v.at[b, pl.ds(c * _CHUNK, _CHUNK)]],
                rows_v.at[buf], g_sem[buf])

        _LOOK = 2
        for i in range(_LOOK):
            start_gather(i, i)
        pltpu.make_async_copy(
            pe_hbm.at[pl.ds(0, _S_PER_W * _D // 2)], pe_v, pe_sem).wait()

        @pl.loop(0, _NBLK, step=_NBUF)
        def _(i0):
            for u in range(_NBUF):
                i = i0 + u
                buf = u
                nbuf = (u + 2) % _NBUF
                c = i >> 2
                b = i & 3

                @pl.when((i >= 2) & (i < _NBLK - 2))
                def _():
                    pltpu.make_async_copy(
                        rows_v.at[nbuf], out_hbm.at[pl.ds(0, _CHUNK)],
                        o_sem[nbuf]).wait()

                @pl.when(i < _NBLK - 2)
                def _():
                    start_gather(i + 2, nbuf)

                pltpu.make_async_copy(
                    table_hbm.at[idx_v.at[0, pl.ds(0, _CHUNK)]],
                    rows_v.at[buf], g_sem[buf]).wait()

                @plsc.parallel_loop(0, _CHUNK, unroll=2)
                def _(r):
                    pe_off = (c * _CHUNK + r) * (_D // 2)
                    for j in range(_D // 32):
                        off = pl.multiple_of(pe_off + j * 16, 8)
                        ab32 = pe_v[pl.ds(off, 16)]
                        ab = plsc.bitcast(ab32, jnp.bfloat16)
                        lo, hi = plsc.unpack(
                            ab, format=plsc.PackFormat.INTERLEAVED)
                        plsc.addupdate(
                            rows_v.at[buf, r, pl.ds(j * 32, _LANES)], lo)
                        plsc.addupdate(
                            rows_v.at[buf, r, pl.ds(j * 32 + 16, _LANES)], hi)

                row0 = b * _S + s_base + c * _CHUNK
                pltpu.async_copy(
                    rows_v.at[buf], out_hbm.at[pl.ds(row0, _CHUNK)],
                    o_sem[buf])

        for buf in range(_NBUF):
            pltpu.make_async_copy(
                rows_v.at[buf], out_hbm.at[pl.ds(0, _CHUNK)], o_sem[buf]
            ).wait()

    return run(idx, table, pe)


def kernel(x, table):
    idx = x.astype(jnp.int32)
    out = _embed(idx, table, _pe_on_device())
    return out.reshape(_B, _S, _D)
